# baseline (speedup 1.0000x reference)
import functools

import jax
import jax.numpy as jnp
from jax import lax
from jax.experimental import pallas as pl
from jax.experimental.pallas import tpu as pltpu
from jax.experimental.pallas import tpu_sc as plsc

_NC = 2
_NS = 16
_NW = _NC * _NS
_EB = 128
_ROWS = 1000


def _make_propagate(n_pad, C, KL):
    rpt = n_pad // _NS
    kb = KL + 2
    mesh = plsc.VectorSubcoreMesh(
        core_axis_name="c", subcore_axis_name="s",
        num_cores=_NC, num_subcores=_NS)

    @functools.partial(
        pl.kernel,
        out_type=jax.ShapeDtypeStruct((_NC, n_pad, C), jnp.float32),
        mesh=mesh,
        compiler_params=pltpu.CompilerParams(use_tc_tiling_on_sc=False),
        scratch_types=[
            pltpu.VMEM((kb, _EB), jnp.int32),
            pltpu.VMEM((kb, _EB), jnp.int32),
            pltpu.VMEM((_EB, C), jnp.float32),
            pltpu.VMEM((rpt, C), jnp.float32),
            pltpu.VMEM_SHARED((n_pad, C), jnp.float32),
            pltpu.SemaphoreType.DMA,
        ],
    )
    def prop(t_hbm, src_hbm, dst_hbm, zeros_hbm, out_hbm,
             src_v, dst_v, buf_a, row_v, acc, sem_a):
        c = lax.axis_index("c")
        s = lax.axis_index("s")
        w = s * _NC + c
        pltpu.sync_copy(src_hbm.at[w], src_v)
        pltpu.sync_copy(dst_hbm.at[w], dst_v)
        r0 = s * rpt
        pltpu.sync_copy(zeros_hbm, row_v)
        pltpu.sync_copy(row_v, acc.at[pl.ds(r0, rpt)])
        plsc.subcore_barrier()

        def body(j, carry):
            pltpu.async_copy(t_hbm.at[src_v.at[j]], buf_a, sem_a).wait()
            pltpu.sync_copy(buf_a, acc.at[dst_v.at[j]], add=True)
            return carry

        lax.fori_loop(0, KL, body, 0)
        plsc.subcore_barrier()
        pltpu.sync_copy(acc.at[pl.ds(r0, rpt)], row_v)
        pltpu.sync_copy(row_v, out_hbm.at[c, pl.ds(r0, rpt)])

    return prop


def _make_count(n_pad, KL):
    C = 16
    rpt = n_pad // _NS
    kb = KL + 2
    mesh = plsc.VectorSubcoreMesh(
        core_axis_name="c", subcore_axis_name="s",
        num_cores=_NC, num_subcores=_NS)

    @functools.partial(
        pl.kernel,
        out_type=jax.ShapeDtypeStruct((_NC, n_pad, C), jnp.float32),
        mesh=mesh,
        compiler_params=pltpu.CompilerParams(use_tc_tiling_on_sc=False),
        scratch_types=[
            pltpu.VMEM((kb, _EB), jnp.int32),
            pltpu.VMEM((_EB, C), jnp.float32),
            pltpu.VMEM((rpt, C), jnp.float32),
            pltpu.VMEM_SHARED((n_pad, C), jnp.float32),
        ],
    )
    def cnt(ones_hbm, dst_hbm, zeros_hbm, out_hbm, dst_v, buf, row_v, acc):
        c = lax.axis_index("c")
        s = lax.axis_index("s")
        w = s * _NC + c
        pltpu.sync_copy(dst_hbm.at[w], dst_v)
        pltpu.sync_copy(ones_hbm, buf)
        r0 = s * rpt
        pltpu.sync_copy(zeros_hbm, row_v)
        pltpu.sync_copy(row_v, acc.at[pl.ds(r0, rpt)])
        plsc.subcore_barrier()

        def body(j, carry):
            pltpu.sync_copy(buf, acc.at[dst_v.at[j]], add=True)
            return carry

        lax.fori_loop(0, KL, body, 0)
        plsc.subcore_barrier()
        pltpu.sync_copy(acc.at[pl.ds(r0, rpt)], row_v)
        pltpu.sync_copy(row_v, out_hbm.at[c, pl.ds(r0, rpt)])

    return cnt


def _mm_scale(h, wc, dinv):
    n, d = h.shape
    hdim = wc.shape[1]

    def kern(h_ref, w_ref, d_ref, o_ref):
        p = jnp.dot(h_ref[...], w_ref[...], preferred_element_type=jnp.float32)
        col = lax.broadcasted_iota(jnp.int32, (_ROWS, hdim), 1)
        o_ref[...] = p * jnp.where(col < 32, 1.0, d_ref[...])

    return pl.pallas_call(
        kern,
        grid=(n // _ROWS,),
        in_specs=[
            pl.BlockSpec((_ROWS, d), lambda i: (i, 0)),
            pl.BlockSpec((d, hdim), lambda i: (0, 0)),
            pl.BlockSpec((_ROWS, 1), lambda i: (i, 0)),
        ],
        out_specs=pl.BlockSpec((_ROWS, hdim), lambda i: (i, 0)),
        out_shape=jax.ShapeDtypeStruct((n, hdim), jnp.float32),
    )(h, wc, dinv)


def _mid(p0, p1, t, dinv):
    n, cdim = t.shape

    def kern(a_ref, b_ref, t_ref, d_ref, o_ref):
        d = d_ref[...]
        u = (a_ref[...] + b_ref[...] + t_ref[...]) * d
        col = lax.broadcasted_iota(jnp.int32, (_ROWS, cdim), 1)
        o_ref[...] = u * jnp.where(col < 32, 1.0, d)

    spec = pl.BlockSpec((_ROWS, cdim), lambda i: (i, 0))
    return pl.pallas_call(
        kern,
        grid=(n // _ROWS,),
        in_specs=[spec, spec, spec, pl.BlockSpec((_ROWS, 1), lambda i: (i, 0))],
        out_specs=spec,
        out_shape=jax.ShapeDtypeStruct((n, cdim), jnp.float32),
    )(p0, p1, t, dinv)


def _fin(q0, q1, t2, p_first, u_first, bc, dinv):
    n, cdim = t2.shape

    def kern(q0r, q1r, t2r, pr, ur, br, dr, o_ref):
        v = (q0r[...] + q1r[...] + t2r[...]) * dr[...]
        cat = jnp.concatenate([pr[...], ur[...], v], axis=1)
        o_ref[...] = jnp.tanh(cat + br[...])

    spec32 = pl.BlockSpec((_ROWS, cdim), lambda i: (i, 0))
    return pl.pallas_call(
        kern,
        grid=(n // _ROWS,),
        in_specs=[
            spec32, spec32, spec32, spec32, spec32,
            pl.BlockSpec((1, 3 * cdim), lambda i: (0, 0)),
            pl.BlockSpec((_ROWS, 1), lambda i: (i, 0)),
        ],
        out_specs=pl.BlockSpec((_ROWS, 3 * cdim), lambda i: (i, 0)),
        out_shape=jax.ShapeDtypeStruct((n, 3 * cdim), jnp.float32),
    )(q0, q1, t2, p_first, u_first, bc, dinv)


def _dinv_from_counts(c0, c1):
    n = c0.shape[0]

    def kern(a_ref, b_ref, o_ref):
        o_ref[...] = lax.rsqrt(a_ref[...] + b_ref[...] + 1.0)

    spec = pl.BlockSpec((_ROWS, 1), lambda i: (i, 0))
    return pl.pallas_call(
        kern,
        grid=(n // _ROWS,),
        in_specs=[spec, spec],
        out_specs=spec,
        out_shape=jax.ShapeDtypeStruct((n, 1), jnp.float32),
    )(c0, c1)


def kernel(x, params, edge_index):
    n, _ = x.shape
    e = edge_index.shape[1]
    n_pad = -(-(n + 1) // (_NS * 8)) * (_NS * 8)
    k = -(-e // (_NW * _EB))
    kl = k + (k % 2)
    e_pad = kl * _NW * _EB
    pad_rows = n_pad - n

    pad_dst = n + jnp.arange(e_pad - e, dtype=jnp.int32) % pad_rows
    src = jnp.concatenate(
        [edge_index[0], jnp.zeros((e_pad - e,), jnp.int32)]).reshape(_NW, kl, _EB)
    dst = jnp.concatenate(
        [edge_index[1], pad_dst]).reshape(_NW, kl, _EB)
    lookahead = n + jnp.arange(2 * _EB, dtype=jnp.int32) % pad_rows
    src = jnp.concatenate([src, jnp.zeros((_NW, 2, _EB), jnp.int32)], axis=1)
    dst = jnp.concatenate(
        [dst, jnp.broadcast_to(lookahead.reshape(1, 2, _EB), (_NW, 2, _EB))],
        axis=1)

    rpt = n_pad // _NS
    z16 = jnp.zeros((rpt, 16), jnp.float32)
    z32 = jnp.zeros((rpt, 32), jnp.float32)
    z64 = jnp.zeros((rpt, 64), jnp.float32)
    prop32 = _make_propagate(n_pad, 32, kl)
    prop64 = _make_propagate(n_pad, 64, kl)

    cnt = _make_propagate(n_pad, 16, kl)(
        jnp.ones((n_pad, 16), jnp.float32), src, dst, z16)
    dinv = _dinv_from_counts(cnt[0, :n, :1], cnt[1, :n, :1])

    h = x
    for lname in ("1", "2", "3"):
        wc = jnp.concatenate(params["W" + lname], axis=1)
        bc = jnp.concatenate(params["b" + lname]).reshape(1, -1)
        P = _mm_scale(h, wc, dinv)
        t = P[:, 32:]
        t_pad = jnp.concatenate([t, jnp.zeros((pad_rows, 64), jnp.float32)], 0)
        pp = prop64(t_pad, src, dst, z64)
        U = _mid(pp[0, :n], pp[1, :n], t, dinv)
        t2 = U[:, 32:]
        t2_pad = jnp.concatenate([t2, jnp.zeros((pad_rows, 32), jnp.float32)], 0)
        qq = prop32(t2_pad, src, dst, z32)
        h = _fin(qq[0, :n], qq[1, :n], t2, P[:, :32], U[:, :32], bc, dinv)
    return h


# speedup vs baseline: 1.3084x; 1.3084x over previous
import functools

import jax
import jax.numpy as jnp
from jax import lax
from jax.experimental import pallas as pl
from jax.experimental.pallas import tpu as pltpu
from jax.experimental.pallas import tpu_sc as plsc

_NC = 2
_NS = 16
_NW = _NC * _NS
_EB = 128
_ROWS = 1000


def _make_propagate(n_pad, C, KL):
    rpt = n_pad // _NS
    kb = KL
    mesh = plsc.VectorSubcoreMesh(
        core_axis_name="c", subcore_axis_name="s",
        num_cores=_NC, num_subcores=_NS)

    @functools.partial(
        pl.kernel,
        out_type=jax.ShapeDtypeStruct((_NC, n_pad, C), jnp.float32),
        mesh=mesh,
        compiler_params=pltpu.CompilerParams(use_tc_tiling_on_sc=False),
        scratch_types=[
            pltpu.VMEM((kb, _EB), jnp.int32),
            pltpu.VMEM((kb, _EB), jnp.int32),
            pltpu.VMEM((_EB, C), jnp.float32),
            pltpu.VMEM((rpt, C), jnp.float32),
            pltpu.VMEM_SHARED((n_pad, C), jnp.float32),
            pltpu.SemaphoreType.DMA,
        ],
    )
    def prop(t_hbm, src_hbm, dst_hbm, zeros_hbm, out_hbm,
             src_v, dst_v, buf_a, row_v, acc, sem_a):
        c = lax.axis_index("c")
        s = lax.axis_index("s")
        w = s * _NC + c
        pltpu.sync_copy(src_hbm.at[w], src_v)
        pltpu.sync_copy(dst_hbm.at[w], dst_v)
        r0 = s * rpt
        pltpu.sync_copy(zeros_hbm, row_v)
        pltpu.sync_copy(row_v, acc.at[pl.ds(r0, rpt)])
        plsc.subcore_barrier()

        def body(j, carry):
            pltpu.async_copy(t_hbm.at[src_v.at[j]], buf_a, sem_a).wait()
            pltpu.sync_copy(buf_a, acc.at[dst_v.at[j]], add=True)
            return carry

        lax.fori_loop(0, KL, body, 0)
        plsc.subcore_barrier()
        pltpu.sync_copy(acc.at[pl.ds(r0, rpt)], row_v)
        pltpu.sync_copy(row_v, out_hbm.at[c, pl.ds(r0, rpt)])

    return prop


def _make_count(n_pad, KL):
    C = 16
    rpt = n_pad // _NS
    kb = KL
    mesh = plsc.VectorSubcoreMesh(
        core_axis_name="c", subcore_axis_name="s",
        num_cores=_NC, num_subcores=_NS)

    @functools.partial(
        pl.kernel,
        out_type=jax.ShapeDtypeStruct((_NC, n_pad, C), jnp.float32),
        mesh=mesh,
        compiler_params=pltpu.CompilerParams(use_tc_tiling_on_sc=False),
        scratch_types=[
            pltpu.VMEM((kb, _EB), jnp.int32),
            pltpu.VMEM((_EB, C), jnp.float32),
            pltpu.VMEM((rpt, C), jnp.float32),
            pltpu.VMEM_SHARED((n_pad, C), jnp.float32),
        ],
    )
    def cnt(ones_hbm, dst_hbm, zeros_hbm, out_hbm, dst_v, buf, row_v, acc):
        c = lax.axis_index("c")
        s = lax.axis_index("s")
        w = s * _NC + c
        pltpu.sync_copy(dst_hbm.at[w], dst_v)
        pltpu.sync_copy(ones_hbm, buf)
        r0 = s * rpt
        pltpu.sync_copy(zeros_hbm, row_v)
        pltpu.sync_copy(row_v, acc.at[pl.ds(r0, rpt)])
        plsc.subcore_barrier()

        def body(j, carry):
            pltpu.sync_copy(buf, acc.at[dst_v.at[j]], add=True)
            return carry

        lax.fori_loop(0, KL, body, 0)
        plsc.subcore_barrier()
        pltpu.sync_copy(acc.at[pl.ds(r0, rpt)], row_v)
        pltpu.sync_copy(row_v, out_hbm.at[c, pl.ds(r0, rpt)])

    return cnt


def _mm_scale(h, wc, dinv):
    n, d = h.shape
    hdim = wc.shape[1]

    def kern(h_ref, w_ref, d_ref, o_ref):
        p = jnp.dot(h_ref[...], w_ref[...], preferred_element_type=jnp.float32)
        col = lax.broadcasted_iota(jnp.int32, (_ROWS, hdim), 1)
        o_ref[...] = p * jnp.where(col < 32, 1.0, d_ref[...])

    return pl.pallas_call(
        kern,
        grid=(n // _ROWS,),
        in_specs=[
            pl.BlockSpec((_ROWS, d), lambda i: (i, 0)),
            pl.BlockSpec((d, hdim), lambda i: (0, 0)),
            pl.BlockSpec((_ROWS, 1), lambda i: (i, 0)),
        ],
        out_specs=pl.BlockSpec((_ROWS, hdim), lambda i: (i, 0)),
        out_shape=jax.ShapeDtypeStruct((n, hdim), jnp.float32),
    )(h, wc, dinv)


def _mid(p0, p1, t, dinv):
    n, cdim = t.shape

    def kern(a_ref, b_ref, t_ref, d_ref, o_ref):
        d = d_ref[...]
        u = (a_ref[...] + b_ref[...] + t_ref[...]) * d
        col = lax.broadcasted_iota(jnp.int32, (_ROWS, cdim), 1)
        o_ref[...] = u * jnp.where(col < 32, 1.0, d)

    spec = pl.BlockSpec((_ROWS, cdim), lambda i: (i, 0))
    return pl.pallas_call(
        kern,
        grid=(n // _ROWS,),
        in_specs=[spec, spec, spec, pl.BlockSpec((_ROWS, 1), lambda i: (i, 0))],
        out_specs=spec,
        out_shape=jax.ShapeDtypeStruct((n, cdim), jnp.float32),
    )(p0, p1, t, dinv)


def _fin(q0, q1, t2, p_first, u_first, bc, dinv):
    n, cdim = t2.shape

    def kern(q0r, q1r, t2r, pr, ur, br, dr, o_ref):
        v = (q0r[...] + q1r[...] + t2r[...]) * dr[...]
        cat = jnp.concatenate([pr[...], ur[...], v], axis=1)
        o_ref[...] = jnp.tanh(cat + br[...])

    spec32 = pl.BlockSpec((_ROWS, cdim), lambda i: (i, 0))
    return pl.pallas_call(
        kern,
        grid=(n // _ROWS,),
        in_specs=[
            spec32, spec32, spec32, spec32, spec32,
            pl.BlockSpec((1, 3 * cdim), lambda i: (0, 0)),
            pl.BlockSpec((_ROWS, 1), lambda i: (i, 0)),
        ],
        out_specs=pl.BlockSpec((_ROWS, 3 * cdim), lambda i: (i, 0)),
        out_shape=jax.ShapeDtypeStruct((n, 3 * cdim), jnp.float32),
    )(q0, q1, t2, p_first, u_first, bc, dinv)


def _dinv_from_counts(c0, c1):
    n = c0.shape[0]

    def kern(a_ref, b_ref, o_ref):
        o_ref[...] = lax.rsqrt(a_ref[...] + b_ref[...] + 1.0)

    spec = pl.BlockSpec((_ROWS, 1), lambda i: (i, 0))
    return pl.pallas_call(
        kern,
        grid=(n // _ROWS,),
        in_specs=[spec, spec],
        out_specs=spec,
        out_shape=jax.ShapeDtypeStruct((n, 1), jnp.float32),
    )(c0, c1)


def kernel(x, params, edge_index):
    n, _ = x.shape
    e = edge_index.shape[1]
    n_pad = -(-(n + 1) // (_NS * 8)) * (_NS * 8)
    k = -(-e // (_NW * _EB))
    kl = k
    e_pad = kl * _NW * _EB
    pad_rows = n_pad - n

    pad_dst = n + jnp.arange(e_pad - e, dtype=jnp.int32) % pad_rows
    src = jnp.concatenate(
        [edge_index[0], jnp.zeros((e_pad - e,), jnp.int32)]).reshape(_NW, kl, _EB)
    dst = jnp.concatenate(
        [edge_index[1], pad_dst]).reshape(_NW, kl, _EB)

    rpt = n_pad // _NS
    z16 = jnp.zeros((rpt, 16), jnp.float32)
    z32 = jnp.zeros((rpt, 32), jnp.float32)
    z64 = jnp.zeros((rpt, 64), jnp.float32)
    prop32 = _make_propagate(n_pad, 32, kl)
    prop64 = _make_propagate(n_pad, 64, kl)

    cnt = _make_propagate(n_pad, 16, kl)(
        jnp.ones((n_pad, 16), jnp.float32), src, dst, z16)
    dinv = _dinv_from_counts(cnt[0, :n, :1], cnt[1, :n, :1])

    h = x
    for lname in ("1", "2", "3"):
        wc = jnp.concatenate(params["W" + lname], axis=1)
        bc = jnp.concatenate(params["b" + lname]).reshape(1, -1)
        P = _mm_scale(h, wc, dinv)
        t = P[:, 32:]
        t_pad = jnp.concatenate([t, jnp.zeros((pad_rows, 64), jnp.float32)], 0)
        pp = prop64(t_pad, src, dst, z64)
        U = _mid(pp[0, :n], pp[1, :n], t, dinv)
        t2 = U[:, 32:]
        t2_pad = jnp.concatenate([t2, jnp.zeros((pad_rows, 32), jnp.float32)], 0)
        qq = prop32(t2_pad, src, dst, z32)
        h = _fin(qq[0, :n], qq[1, :n], t2, P[:, :32], U[:, :32], bc, dinv)
    return h


# speedup vs baseline: 1.7295x; 1.3219x over previous
import functools

import jax
import jax.numpy as jnp
from jax import lax
from jax.experimental import pallas as pl
from jax.experimental.pallas import tpu as pltpu
from jax.experimental.pallas import tpu_sc as plsc

_NC = 2
_NS = 16
_NW = _NC * _NS
_EB = 128
_ROWS = 1000


def _make_propagate(n_pad, C, KL):
    rpt = n_pad // _NS
    kb = KL
    mesh = plsc.VectorSubcoreMesh(
        core_axis_name="c", subcore_axis_name="s",
        num_cores=_NC, num_subcores=_NS)

    @functools.partial(
        pl.kernel,
        out_type=jax.ShapeDtypeStruct((_NC, n_pad, C), jnp.float32),
        mesh=mesh,
        compiler_params=pltpu.CompilerParams(use_tc_tiling_on_sc=False),
        scratch_types=[
            pltpu.VMEM((kb, _EB), jnp.int32),
            pltpu.VMEM((kb, _EB), jnp.int32),
            pltpu.VMEM((_EB, C), jnp.float32),
            pltpu.VMEM((rpt, C), jnp.float32),
            pltpu.VMEM_SHARED((n_pad, C), jnp.float32),
            pltpu.SemaphoreType.DMA,
        ],
    )
    def prop(t_hbm, src_hbm, dst_hbm, zeros_hbm, out_hbm,
             src_v, dst_v, buf_a, row_v, acc, sem_a):
        c = lax.axis_index("c")
        s = lax.axis_index("s")
        w = s * _NC + c
        pltpu.sync_copy(src_hbm.at[w], src_v)
        pltpu.sync_copy(dst_hbm.at[w], dst_v)
        r0 = s * rpt
        pltpu.sync_copy(zeros_hbm, row_v)
        pltpu.sync_copy(row_v, acc.at[pl.ds(r0, rpt)])
        plsc.subcore_barrier()

        def body(j, carry):
            pltpu.async_copy(t_hbm.at[src_v.at[j]], buf_a, sem_a).wait()
            pltpu.sync_copy(buf_a, acc.at[dst_v.at[j]], add=True)
            return carry

        lax.fori_loop(0, KL, body, 0)
        plsc.subcore_barrier()
        pltpu.sync_copy(acc.at[pl.ds(r0, rpt)], row_v)
        pltpu.sync_copy(row_v, out_hbm.at[c, pl.ds(r0, rpt)])

    return prop


def _make_count(n_pad, KL):
    C = 16
    rpt = n_pad // _NS
    kb = KL
    mesh = plsc.VectorSubcoreMesh(
        core_axis_name="c", subcore_axis_name="s",
        num_cores=_NC, num_subcores=_NS)

    @functools.partial(
        pl.kernel,
        out_type=jax.ShapeDtypeStruct((_NC, n_pad, C), jnp.float32),
        mesh=mesh,
        compiler_params=pltpu.CompilerParams(use_tc_tiling_on_sc=False),
        scratch_types=[
            pltpu.VMEM((kb, _EB), jnp.int32),
            pltpu.VMEM((_EB, C), jnp.float32),
            pltpu.VMEM((rpt, C), jnp.float32),
            pltpu.VMEM_SHARED((n_pad, C), jnp.float32),
        ],
    )
    def cnt(ones_hbm, dst_hbm, zeros_hbm, out_hbm, dst_v, buf, row_v, acc):
        c = lax.axis_index("c")
        s = lax.axis_index("s")
        w = s * _NC + c
        pltpu.sync_copy(dst_hbm.at[w], dst_v)
        pltpu.sync_copy(ones_hbm, buf)
        r0 = s * rpt
        pltpu.sync_copy(zeros_hbm, row_v)
        pltpu.sync_copy(row_v, acc.at[pl.ds(r0, rpt)])
        plsc.subcore_barrier()

        def body(j, carry):
            pltpu.sync_copy(buf, acc.at[dst_v.at[j]], add=True)
            return carry

        lax.fori_loop(0, KL, body, 0)
        plsc.subcore_barrier()
        pltpu.sync_copy(acc.at[pl.ds(r0, rpt)], row_v)
        pltpu.sync_copy(row_v, out_hbm.at[c, pl.ds(r0, rpt)])

    return cnt


def _mm_scale(h, wc, dinv):
    n, d = h.shape
    hdim = wc.shape[1]

    def kern(h_ref, w_ref, d_ref, o_ref):
        p = jnp.dot(h_ref[...], w_ref[...], preferred_element_type=jnp.float32)
        col = lax.broadcasted_iota(jnp.int32, (_ROWS, hdim), 1)
        o_ref[...] = p * jnp.where(col < 32, 1.0, d_ref[...])

    return pl.pallas_call(
        kern,
        grid=(n // _ROWS,),
        in_specs=[
            pl.BlockSpec((_ROWS, d), lambda i: (i, 0)),
            pl.BlockSpec((d, hdim), lambda i: (0, 0)),
            pl.BlockSpec((_ROWS, 1), lambda i: (i, 0)),
        ],
        out_specs=pl.BlockSpec((_ROWS, hdim), lambda i: (i, 0)),
        out_shape=jax.ShapeDtypeStruct((n, hdim), jnp.float32),
    )(h, wc, dinv)


def _mid(p0, p1, t, dinv):
    n, cdim = t.shape

    def kern(a_ref, b_ref, t_ref, d_ref, o_ref):
        d = d_ref[...]
        u = (a_ref[...] + b_ref[...] + t_ref[...]) * d
        col = lax.broadcasted_iota(jnp.int32, (_ROWS, cdim), 1)
        o_ref[...] = u * jnp.where(col < 32, 1.0, d)

    spec = pl.BlockSpec((_ROWS, cdim), lambda i: (i, 0))
    return pl.pallas_call(
        kern,
        grid=(n // _ROWS,),
        in_specs=[spec, spec, spec, pl.BlockSpec((_ROWS, 1), lambda i: (i, 0))],
        out_specs=spec,
        out_shape=jax.ShapeDtypeStruct((n, cdim), jnp.float32),
    )(p0, p1, t, dinv)


def _fin(q0, q1, t2, p_first, u_first, bc, dinv):
    n, cdim = t2.shape

    def kern(q0r, q1r, t2r, pr, ur, br, dr, o_ref):
        v = (q0r[...] + q1r[...] + t2r[...]) * dr[...]
        cat = jnp.concatenate([pr[...], ur[...], v], axis=1)
        o_ref[...] = jnp.tanh(cat + br[...])

    spec32 = pl.BlockSpec((_ROWS, cdim), lambda i: (i, 0))
    return pl.pallas_call(
        kern,
        grid=(n // _ROWS,),
        in_specs=[
            spec32, spec32, spec32, spec32, spec32,
            pl.BlockSpec((1, 3 * cdim), lambda i: (0, 0)),
            pl.BlockSpec((_ROWS, 1), lambda i: (i, 0)),
        ],
        out_specs=pl.BlockSpec((_ROWS, 3 * cdim), lambda i: (i, 0)),
        out_shape=jax.ShapeDtypeStruct((n, 3 * cdim), jnp.float32),
    )(q0, q1, t2, p_first, u_first, bc, dinv)


def _dinv_from_counts(c0, c1):
    n = c0.shape[0]

    def kern(a_ref, b_ref, o_ref):
        o_ref[...] = lax.rsqrt(a_ref[...] + b_ref[...] + 1.0)

    spec = pl.BlockSpec((_ROWS, 1), lambda i: (i, 0))
    return pl.pallas_call(
        kern,
        grid=(n // _ROWS,),
        in_specs=[spec, spec],
        out_specs=spec,
        out_shape=jax.ShapeDtypeStruct((n, 1), jnp.float32),
    )(c0, c1)


def kernel(x, params, edge_index):
    n, _ = x.shape
    e = edge_index.shape[1]
    n_pad = -(-(n + 1) // (_NS * 8)) * (_NS * 8)
    k = -(-e // (_NW * _EB))
    kl = k
    e_pad = kl * _NW * _EB
    pad_rows = n_pad - n

    pad_src = jnp.arange(e_pad - e, dtype=jnp.int32) % n
    pad_dst = n + jnp.arange(e_pad - e, dtype=jnp.int32) % pad_rows
    src = jnp.concatenate(
        [edge_index[0], pad_src]).reshape(_NW, kl, _EB)
    dst = jnp.concatenate(
        [edge_index[1], pad_dst]).reshape(_NW, kl, _EB)

    rpt = n_pad // _NS
    z16 = jnp.zeros((rpt, 16), jnp.float32)
    z32 = jnp.zeros((rpt, 32), jnp.float32)
    z64 = jnp.zeros((rpt, 64), jnp.float32)
    prop32 = _make_propagate(n_pad, 32, kl)
    prop64 = _make_propagate(n_pad, 64, kl)

    cnt = _make_propagate(n_pad, 16, kl)(
        jnp.ones((n_pad, 16), jnp.float32), src, dst, z16)
    dinv = _dinv_from_counts(cnt[0, :n, :1], cnt[1, :n, :1])

    h = x
    for lname in ("1", "2", "3"):
        wc = jnp.concatenate(params["W" + lname], axis=1)
        bc = jnp.concatenate(params["b" + lname]).reshape(1, -1)
        P = _mm_scale(h, wc, dinv)
        t = P[:, 32:]
        t_pad = jnp.concatenate([t, jnp.zeros((pad_rows, 64), jnp.float32)], 0)
        pp = prop64(t_pad, src, dst, z64)
        U = _mid(pp[0, :n], pp[1, :n], t, dinv)
        t2 = U[:, 32:]
        t2_pad = jnp.concatenate([t2, jnp.zeros((pad_rows, 32), jnp.float32)], 0)
        qq = prop32(t2_pad, src, dst, z32)
        h = _fin(qq[0, :n], qq[1, :n], t2, P[:, :32], U[:, :32], bc, dinv)
    return h


# speedup vs baseline: 2.3525x; 1.3603x over previous
import functools

import jax
import jax.numpy as jnp
from jax import lax
from jax.experimental import pallas as pl
from jax.experimental.pallas import tpu as pltpu
from jax.experimental.pallas import tpu_sc as plsc

_NC = 2
_NS = 16
_NW = _NC * _NS
_EB = 128
_ROWS = 1000


def _make_propagate(n_pad, C, KL):
    rpt = n_pad // _NS
    kb = KL + 2
    mesh = plsc.VectorSubcoreMesh(
        core_axis_name="c", subcore_axis_name="s",
        num_cores=_NC, num_subcores=_NS)

    @functools.partial(
        pl.kernel,
        out_type=jax.ShapeDtypeStruct((_NC, n_pad, C), jnp.float32),
        mesh=mesh,
        compiler_params=pltpu.CompilerParams(use_tc_tiling_on_sc=False),
        scratch_types=[
            pltpu.VMEM((kb, _EB), jnp.int32),
            pltpu.VMEM((kb, _EB), jnp.int32),
            pltpu.VMEM((_EB, C), jnp.float32),
            pltpu.VMEM((_EB, C), jnp.float32),
            pltpu.VMEM((rpt, C), jnp.float32),
            pltpu.VMEM_SHARED((n_pad, C), jnp.float32),
            pltpu.SemaphoreType.DMA,
            pltpu.SemaphoreType.DMA,
        ],
    )
    def prop(t_hbm, src_hbm, dst_hbm, zeros_hbm, out_hbm,
             src_v, dst_v, buf_a, buf_b, row_v, acc, sem_a, sem_b):
        c = lax.axis_index("c")
        s = lax.axis_index("s")
        w = s * _NC + c
        pltpu.sync_copy(src_hbm.at[w], src_v)
        pltpu.sync_copy(dst_hbm.at[w], dst_v)
        r0 = s * rpt
        pltpu.sync_copy(zeros_hbm, row_v)
        pltpu.sync_copy(row_v, acc.at[pl.ds(r0, rpt)])
        plsc.subcore_barrier()

        pltpu.async_copy(t_hbm.at[src_v.at[0]], buf_a, sem_a)

        def body(i, carry):
            j = 2 * i
            pltpu.async_copy(t_hbm.at[src_v.at[j + 1]], buf_b, sem_b)
            pltpu.make_async_copy(t_hbm.at[src_v.at[j]], buf_a, sem_a).wait()
            pltpu.sync_copy(buf_a, acc.at[dst_v.at[j]], add=True)
            pltpu.async_copy(t_hbm.at[src_v.at[j + 2]], buf_a, sem_a)
            pltpu.make_async_copy(t_hbm.at[src_v.at[j + 1]], buf_b, sem_b).wait()
            pltpu.sync_copy(buf_b, acc.at[dst_v.at[j + 1]], add=True)
            return carry

        lax.fori_loop(0, KL // 2, body, 0)
        pltpu.make_async_copy(t_hbm.at[src_v.at[KL]], buf_a, sem_a).wait()
        plsc.subcore_barrier()
        pltpu.sync_copy(acc.at[pl.ds(r0, rpt)], row_v)
        pltpu.sync_copy(row_v, out_hbm.at[c, pl.ds(r0, rpt)])

    return prop


def _make_count(n_pad, KL):
    C = 16
    rpt = n_pad // _NS
    kb = KL + 2
    mesh = plsc.VectorSubcoreMesh(
        core_axis_name="c", subcore_axis_name="s",
        num_cores=_NC, num_subcores=_NS)

    @functools.partial(
        pl.kernel,
        out_type=jax.ShapeDtypeStruct((_NC, n_pad, C), jnp.float32),
        mesh=mesh,
        compiler_params=pltpu.CompilerParams(use_tc_tiling_on_sc=False),
        scratch_types=[
            pltpu.VMEM((kb, _EB), jnp.int32),
            pltpu.VMEM((_EB, C), jnp.float32),
            pltpu.VMEM((rpt, C), jnp.float32),
            pltpu.VMEM_SHARED((n_pad, C), jnp.float32),
        ],
    )
    def cnt(ones_hbm, dst_hbm, zeros_hbm, out_hbm, dst_v, buf, row_v, acc):
        c = lax.axis_index("c")
        s = lax.axis_index("s")
        w = s * _NC + c
        pltpu.sync_copy(dst_hbm.at[w], dst_v)
        pltpu.sync_copy(ones_hbm, buf)
        r0 = s * rpt
        pltpu.sync_copy(zeros_hbm, row_v)
        pltpu.sync_copy(row_v, acc.at[pl.ds(r0, rpt)])
        plsc.subcore_barrier()

        def body(j, carry):
            pltpu.sync_copy(buf, acc.at[dst_v.at[j]], add=True)
            return carry

        lax.fori_loop(0, KL, body, 0)
        plsc.subcore_barrier()
        pltpu.sync_copy(acc.at[pl.ds(r0, rpt)], row_v)
        pltpu.sync_copy(row_v, out_hbm.at[c, pl.ds(r0, rpt)])

    return cnt


def _mm_scale(h, wc, dinv):
    n, d = h.shape
    hdim = wc.shape[1]

    def kern(h_ref, w_ref, d_ref, o_ref):
        p = jnp.dot(h_ref[...], w_ref[...], preferred_element_type=jnp.float32)
        col = lax.broadcasted_iota(jnp.int32, (_ROWS, hdim), 1)
        o_ref[...] = p * jnp.where(col < 32, 1.0, d_ref[...])

    return pl.pallas_call(
        kern,
        grid=(n // _ROWS,),
        in_specs=[
            pl.BlockSpec((_ROWS, d), lambda i: (i, 0)),
            pl.BlockSpec((d, hdim), lambda i: (0, 0)),
            pl.BlockSpec((_ROWS, 1), lambda i: (i, 0)),
        ],
        out_specs=pl.BlockSpec((_ROWS, hdim), lambda i: (i, 0)),
        out_shape=jax.ShapeDtypeStruct((n, hdim), jnp.float32),
    )(h, wc, dinv)


def _mid(p0, p1, t, dinv):
    n, cdim = t.shape

    def kern(a_ref, b_ref, t_ref, d_ref, o_ref):
        d = d_ref[...]
        u = (a_ref[...] + b_ref[...] + t_ref[...]) * d
        col = lax.broadcasted_iota(jnp.int32, (_ROWS, cdim), 1)
        o_ref[...] = u * jnp.where(col < 32, 1.0, d)

    spec = pl.BlockSpec((_ROWS, cdim), lambda i: (i, 0))
    return pl.pallas_call(
        kern,
        grid=(n // _ROWS,),
        in_specs=[spec, spec, spec, pl.BlockSpec((_ROWS, 1), lambda i: (i, 0))],
        out_specs=spec,
        out_shape=jax.ShapeDtypeStruct((n, cdim), jnp.float32),
    )(p0, p1, t, dinv)


def _fin(q0, q1, t2, p_first, u_first, bc, dinv):
    n, cdim = t2.shape

    def kern(q0r, q1r, t2r, pr, ur, br, dr, o_ref):
        v = (q0r[...] + q1r[...] + t2r[...]) * dr[...]
        cat = jnp.concatenate([pr[...], ur[...], v], axis=1)
        o_ref[...] = jnp.tanh(cat + br[...])

    spec32 = pl.BlockSpec((_ROWS, cdim), lambda i: (i, 0))
    return pl.pallas_call(
        kern,
        grid=(n // _ROWS,),
        in_specs=[
            spec32, spec32, spec32, spec32, spec32,
            pl.BlockSpec((1, 3 * cdim), lambda i: (0, 0)),
            pl.BlockSpec((_ROWS, 1), lambda i: (i, 0)),
        ],
        out_specs=pl.BlockSpec((_ROWS, 3 * cdim), lambda i: (i, 0)),
        out_shape=jax.ShapeDtypeStruct((n, 3 * cdim), jnp.float32),
    )(q0, q1, t2, p_first, u_first, bc, dinv)


def _dinv_from_counts(c0, c1):
    n = c0.shape[0]

    def kern(a_ref, b_ref, o_ref):
        o_ref[...] = lax.rsqrt(a_ref[...] + b_ref[...] + 1.0)

    spec = pl.BlockSpec((_ROWS, 1), lambda i: (i, 0))
    return pl.pallas_call(
        kern,
        grid=(n // _ROWS,),
        in_specs=[spec, spec],
        out_specs=spec,
        out_shape=jax.ShapeDtypeStruct((n, 1), jnp.float32),
    )(c0, c1)


def kernel(x, params, edge_index):
    n, _ = x.shape
    e = edge_index.shape[1]
    n_pad = -(-(n + 1) // (_NS * 8)) * (_NS * 8)
    k = -(-e // (_NW * _EB))
    kl = k + (k % 2)
    e_pad = kl * _NW * _EB
    pad_rows = n_pad - n

    pad_src = jnp.arange(e_pad - e, dtype=jnp.int32) % n
    pad_dst = n + jnp.arange(e_pad - e, dtype=jnp.int32) % pad_rows
    src = jnp.concatenate(
        [edge_index[0], pad_src]).reshape(_NW, kl, _EB)
    dst = jnp.concatenate(
        [edge_index[1], pad_dst]).reshape(_NW, kl, _EB)
    la_src = (jnp.arange(2 * _EB, dtype=jnp.int32) % n).reshape(1, 2, _EB)
    la_dst = (n + jnp.arange(2 * _EB, dtype=jnp.int32) % pad_rows).reshape(1, 2, _EB)
    src = jnp.concatenate(
        [src, jnp.broadcast_to(la_src, (_NW, 2, _EB))], axis=1)
    dst = jnp.concatenate(
        [dst, jnp.broadcast_to(la_dst, (_NW, 2, _EB))], axis=1)

    rpt = n_pad // _NS
    z16 = jnp.zeros((rpt, 16), jnp.float32)
    z32 = jnp.zeros((rpt, 32), jnp.float32)
    z64 = jnp.zeros((rpt, 64), jnp.float32)
    prop32 = _make_propagate(n_pad, 32, kl)
    prop64 = _make_propagate(n_pad, 64, kl)

    cnt = _make_propagate(n_pad, 16, kl)(
        jnp.ones((n_pad, 16), jnp.float32), src, dst, z16)
    dinv = _dinv_from_counts(cnt[0, :n, :1], cnt[1, :n, :1])

    h = x
    for lname in ("1", "2", "3"):
        wc = jnp.concatenate(params["W" + lname], axis=1)
        bc = jnp.concatenate(params["b" + lname]).reshape(1, -1)
        P = _mm_scale(h, wc, dinv)
        t = P[:, 32:]
        t_pad = jnp.concatenate([t, jnp.zeros((pad_rows, 64), jnp.float32)], 0)
        pp = prop64(t_pad, src, dst, z64)
        U = _mid(pp[0, :n], pp[1, :n], t, dinv)
        t2 = U[:, 32:]
        t2_pad = jnp.concatenate([t2, jnp.zeros((pad_rows, 32), jnp.float32)], 0)
        qq = prop32(t2_pad, src, dst, z32)
        h = _fin(qq[0, :n], qq[1, :n], t2, P[:, :32], U[:, :32], bc, dinv)
    return h


# speedup vs baseline: 2.8827x; 1.2254x over previous
import functools

import jax
import jax.numpy as jnp
from jax import lax
from jax.experimental import pallas as pl
from jax.experimental.pallas import tpu as pltpu
from jax.experimental.pallas import tpu_sc as plsc

_NC = 2
_NS = 16
_NW = _NC * _NS
_EB = 128
_ROWS = 1000


def _make_propagate(n_pad, C, KL):
    rpt = n_pad // _NS
    kb = KL + 2
    mesh = plsc.VectorSubcoreMesh(
        core_axis_name="c", subcore_axis_name="s",
        num_cores=_NC, num_subcores=_NS)

    @functools.partial(
        pl.kernel,
        out_type=jax.ShapeDtypeStruct((_NC, n_pad, C), jnp.float32),
        mesh=mesh,
        compiler_params=pltpu.CompilerParams(use_tc_tiling_on_sc=False),
        scratch_types=[
            pltpu.VMEM((kb, _EB), jnp.int32),
            pltpu.VMEM((kb, _EB), jnp.int32),
            pltpu.VMEM((_EB, C), jnp.float32),
            pltpu.VMEM((_EB, C), jnp.float32),
            pltpu.VMEM((rpt, C), jnp.float32),
            pltpu.VMEM_SHARED((n_pad, C), jnp.float32),
            pltpu.SemaphoreType.DMA,
            pltpu.SemaphoreType.DMA,
        ],
    )
    def prop(t_hbm, src_hbm, dst_hbm, zeros_hbm, out_hbm,
             src_v, dst_v, buf_a, buf_b, row_v, acc, sem_a, sem_b):
        c = lax.axis_index("c")
        s = lax.axis_index("s")
        w = s * _NC + c
        pltpu.sync_copy(src_hbm.at[w], src_v)
        pltpu.sync_copy(dst_hbm.at[w], dst_v)
        r0 = s * rpt
        pltpu.sync_copy(zeros_hbm, row_v)
        pltpu.sync_copy(row_v, acc.at[pl.ds(r0, rpt)])
        plsc.subcore_barrier()

        pltpu.async_copy(t_hbm.at[src_v.at[0]], buf_a, sem_a)

        def body(i, carry):
            j = 2 * i
            pltpu.async_copy(t_hbm.at[src_v.at[j + 1]], buf_b, sem_b)
            pltpu.make_async_copy(t_hbm.at[src_v.at[j]], buf_a, sem_a).wait()
            pltpu.sync_copy(buf_a, acc.at[dst_v.at[j]], add=True)
            pltpu.async_copy(t_hbm.at[src_v.at[j + 2]], buf_a, sem_a)
            pltpu.make_async_copy(t_hbm.at[src_v.at[j + 1]], buf_b, sem_b).wait()
            pltpu.sync_copy(buf_b, acc.at[dst_v.at[j + 1]], add=True)
            return carry

        lax.fori_loop(0, KL // 2, body, 0)
        pltpu.make_async_copy(t_hbm.at[src_v.at[KL]], buf_a, sem_a).wait()
        plsc.subcore_barrier()
        pltpu.sync_copy(acc.at[pl.ds(r0, rpt)], row_v)
        pltpu.sync_copy(row_v, out_hbm.at[c, pl.ds(r0, rpt)])

    return prop


def _make_count(n_pad, KL):
    C = 16
    rpt = n_pad // _NS
    kb = KL + 2
    mesh = plsc.VectorSubcoreMesh(
        core_axis_name="c", subcore_axis_name="s",
        num_cores=_NC, num_subcores=_NS)

    @functools.partial(
        pl.kernel,
        out_type=jax.ShapeDtypeStruct((_NC, n_pad, C), jnp.float32),
        mesh=mesh,
        compiler_params=pltpu.CompilerParams(use_tc_tiling_on_sc=False),
        scratch_types=[
            pltpu.VMEM((kb, _EB), jnp.int32),
            pltpu.VMEM((_EB, C), jnp.float32),
            pltpu.VMEM((rpt, C), jnp.float32),
            pltpu.VMEM_SHARED((n_pad, C), jnp.float32),
        ],
    )
    def cnt(ones_hbm, dst_hbm, zeros_hbm, out_hbm, dst_v, buf, row_v, acc):
        c = lax.axis_index("c")
        s = lax.axis_index("s")
        w = s * _NC + c
        pltpu.sync_copy(dst_hbm.at[w], dst_v)
        pltpu.sync_copy(ones_hbm, buf)
        r0 = s * rpt
        pltpu.sync_copy(zeros_hbm, row_v)
        pltpu.sync_copy(row_v, acc.at[pl.ds(r0, rpt)])
        plsc.subcore_barrier()

        def body(j, carry):
            pltpu.sync_copy(buf, acc.at[dst_v.at[j]], add=True)
            return carry

        lax.fori_loop(0, KL, body, 0)
        plsc.subcore_barrier()
        pltpu.sync_copy(acc.at[pl.ds(r0, rpt)], row_v)
        pltpu.sync_copy(row_v, out_hbm.at[c, pl.ds(r0, rpt)])

    return cnt


def _dinv_from_counts(c0, c1, n_pad):
    rows = n_pad // 8

    def kern(a_ref, b_ref, o_ref):
        o_ref[...] = lax.rsqrt(a_ref[...][:, :1] + b_ref[...][:, :1] + 1.0)

    spec = pl.BlockSpec((rows, 16), lambda i: (i, 0))
    return pl.pallas_call(
        kern,
        grid=(8,),
        in_specs=[spec, spec],
        out_specs=pl.BlockSpec((rows, 1), lambda i: (i, 0)),
        out_shape=jax.ShapeDtypeStruct((n_pad, 1), jnp.float32),
    )(c0, c1)


def _mm_scale(h, wc, dinv):
    n_pad, d = h.shape
    rows = n_pad // 8

    def kern(h_ref, w_ref, d_ref, p1_ref, t_ref):
        p = jnp.dot(h_ref[...], w_ref[...], preferred_element_type=jnp.float32)
        p1_ref[...] = p[:, :32]
        t_ref[...] = p[:, 32:] * d_ref[...]

    return pl.pallas_call(
        kern,
        grid=(8,),
        in_specs=[
            pl.BlockSpec((rows, d), lambda i: (i, 0)),
            pl.BlockSpec((d, 96), lambda i: (0, 0)),
            pl.BlockSpec((rows, 1), lambda i: (i, 0)),
        ],
        out_specs=[
            pl.BlockSpec((rows, 32), lambda i: (i, 0)),
            pl.BlockSpec((rows, 64), lambda i: (i, 0)),
        ],
        out_shape=[
            jax.ShapeDtypeStruct((n_pad, 32), jnp.float32),
            jax.ShapeDtypeStruct((n_pad, 64), jnp.float32),
        ],
    )(h, wc, dinv)


def _mid(pp, t, dinv):
    n_pad = t.shape[0]
    rows = n_pad // 8

    def kern(pp_ref, t_ref, d_ref, u1_ref, t2_ref):
        d = d_ref[...]
        u = (pp_ref[0] + pp_ref[1] + t_ref[...]) * d
        u1_ref[...] = u[:, :32]
        t2_ref[...] = u[:, 32:] * d

    return pl.pallas_call(
        kern,
        grid=(8,),
        in_specs=[
            pl.BlockSpec((2, rows, 64), lambda i: (0, i, 0)),
            pl.BlockSpec((rows, 64), lambda i: (i, 0)),
            pl.BlockSpec((rows, 1), lambda i: (i, 0)),
        ],
        out_specs=[
            pl.BlockSpec((rows, 32), lambda i: (i, 0)),
            pl.BlockSpec((rows, 32), lambda i: (i, 0)),
        ],
        out_shape=[
            jax.ShapeDtypeStruct((n_pad, 32), jnp.float32),
            jax.ShapeDtypeStruct((n_pad, 32), jnp.float32),
        ],
    )(pp, t, dinv)


def _fin_mm(qq, t2, p1, u1, bc, dinv, wc_next):
    n_pad = t2.shape[0]
    rows = n_pad // 8

    def kern(qq_ref, t2_ref, p1_ref, u1_ref, b_ref, d_ref, w_ref,
             p1n_ref, tn_ref):
        d = d_ref[...]
        v = (qq_ref[0] + qq_ref[1] + t2_ref[...]) * d
        cat = jnp.concatenate([p1_ref[...], u1_ref[...], v], axis=1)
        h = jnp.tanh(cat + b_ref[...])
        p = jnp.dot(h, w_ref[...], preferred_element_type=jnp.float32)
        p1n_ref[...] = p[:, :32]
        tn_ref[...] = p[:, 32:] * d

    spec32 = pl.BlockSpec((rows, 32), lambda i: (i, 0))
    return pl.pallas_call(
        kern,
        grid=(8,),
        in_specs=[
            pl.BlockSpec((2, rows, 32), lambda i: (0, i, 0)),
            spec32, spec32, spec32,
            pl.BlockSpec((1, 96), lambda i: (0, 0)),
            pl.BlockSpec((rows, 1), lambda i: (i, 0)),
            pl.BlockSpec((96, 96), lambda i: (0, 0)),
        ],
        out_specs=[
            pl.BlockSpec((rows, 32), lambda i: (i, 0)),
            pl.BlockSpec((rows, 64), lambda i: (i, 0)),
        ],
        out_shape=[
            jax.ShapeDtypeStruct((n_pad, 32), jnp.float32),
            jax.ShapeDtypeStruct((n_pad, 64), jnp.float32),
        ],
    )(qq, t2, p1, u1, bc, dinv, wc_next)


def _fin(qq, t2, p1, u1, bc, dinv):
    n_pad = t2.shape[0]
    rows = n_pad // 8

    def kern(qq_ref, t2_ref, p1_ref, u1_ref, b_ref, d_ref, o_ref):
        v = (qq_ref[0] + qq_ref[1] + t2_ref[...]) * d_ref[...]
        cat = jnp.concatenate([p1_ref[...], u1_ref[...], v], axis=1)
        o_ref[...] = jnp.tanh(cat + b_ref[...])

    spec32 = pl.BlockSpec((rows, 32), lambda i: (i, 0))
    return pl.pallas_call(
        kern,
        grid=(8,),
        in_specs=[
            pl.BlockSpec((2, rows, 32), lambda i: (0, i, 0)),
            spec32, spec32, spec32,
            pl.BlockSpec((1, 96), lambda i: (0, 0)),
            pl.BlockSpec((rows, 1), lambda i: (i, 0)),
        ],
        out_specs=pl.BlockSpec((rows, 96), lambda i: (i, 0)),
        out_shape=jax.ShapeDtypeStruct((n_pad, 96), jnp.float32),
    )(qq, t2, p1, u1, bc, dinv)


def kernel(x, params, edge_index):
    n, d_in = x.shape
    e = edge_index.shape[1]
    n_pad = -(-(n + 1) // (_NS * 8)) * (_NS * 8)
    k = -(-e // (_NW * _EB))
    kl = k + (k % 2)
    e_pad = kl * _NW * _EB
    pad_rows = n_pad - n

    pad_src = jnp.arange(e_pad - e, dtype=jnp.int32) % n
    pad_dst = n + jnp.arange(e_pad - e, dtype=jnp.int32) % pad_rows
    src = jnp.concatenate(
        [edge_index[0], pad_src]).reshape(_NW, kl, _EB)
    dst = jnp.concatenate(
        [edge_index[1], pad_dst]).reshape(_NW, kl, _EB)
    la_src = (jnp.arange(2 * _EB, dtype=jnp.int32) % n).reshape(1, 2, _EB)
    la_dst = (n + jnp.arange(2 * _EB, dtype=jnp.int32) % pad_rows).reshape(1, 2, _EB)
    src = jnp.concatenate(
        [src, jnp.broadcast_to(la_src, (_NW, 2, _EB))], axis=1)
    dst = jnp.concatenate(
        [dst, jnp.broadcast_to(la_dst, (_NW, 2, _EB))], axis=1)

    rpt = n_pad // _NS
    z16 = jnp.zeros((rpt, 16), jnp.float32)
    z32 = jnp.zeros((rpt, 32), jnp.float32)
    z64 = jnp.zeros((rpt, 64), jnp.float32)
    prop32 = _make_propagate(n_pad, 32, kl)
    prop64 = _make_propagate(n_pad, 64, kl)

    cnt = _make_count(n_pad, kl)(jnp.ones((_EB, 16), jnp.float32), dst, z16)
    dinv = _dinv_from_counts(cnt[0], cnt[1], n_pad)

    x_pad = jnp.concatenate(
        [x, jnp.zeros((pad_rows, d_in), jnp.float32)], axis=0)
    wcs = [jnp.concatenate(params["W" + l], axis=1) for l in ("1", "2", "3")]
    bcs = [jnp.concatenate(params["b" + l]).reshape(1, 96) for l in ("1", "2", "3")]

    p1, t = _mm_scale(x_pad, wcs[0], dinv)
    for li in range(3):
        pp = prop64(t, src, dst, z64)
        u1, t2 = _mid(pp, t, dinv)
        qq = prop32(t2, src, dst, z32)
        if li < 2:
            p1, t = _fin_mm(qq, t2, p1, u1, bcs[li], dinv, wcs[li + 1])
        else:
            h = _fin(qq, t2, p1, u1, bcs[li], dinv)
    return h[:n]


# speedup vs baseline: 2.8960x; 1.0046x over previous
import functools

import jax
import jax.numpy as jnp
from jax import lax
from jax.experimental import pallas as pl
from jax.experimental.pallas import tpu as pltpu
from jax.experimental.pallas import tpu_sc as plsc

_NC = 2
_NS = 16
_NW = _NC * _NS
_EB = 128
_ROWS = 1000


def _make_propagate(n_pad, C, KL):
    rpt = n_pad // _NS
    kb = KL + 2
    mesh = plsc.VectorSubcoreMesh(
        core_axis_name="c", subcore_axis_name="s",
        num_cores=_NC, num_subcores=_NS)

    @functools.partial(
        pl.kernel,
        out_type=jax.ShapeDtypeStruct((_NC, n_pad, C), jnp.float32),
        mesh=mesh,
        compiler_params=pltpu.CompilerParams(use_tc_tiling_on_sc=False),
        scratch_types=[
            pltpu.VMEM((kb, _EB), jnp.int32),
            pltpu.VMEM((kb, _EB), jnp.int32),
            pltpu.VMEM((_EB, C), jnp.float32),
            pltpu.VMEM((_EB, C), jnp.float32),
            pltpu.VMEM((rpt, C), jnp.float32),
            pltpu.VMEM_SHARED((n_pad, C), jnp.float32),
            pltpu.SemaphoreType.DMA,
            pltpu.SemaphoreType.DMA,
        ],
    )
    def prop(t_hbm, src_hbm, dst_hbm, zeros_hbm, out_hbm,
             src_v, dst_v, buf_a, buf_b, row_v, acc, sem_a, sem_b):
        c = lax.axis_index("c")
        s = lax.axis_index("s")
        w = s * _NC + c
        pltpu.sync_copy(src_hbm.at[w], src_v)
        pltpu.sync_copy(dst_hbm.at[w], dst_v)
        r0 = s * rpt
        pltpu.sync_copy(zeros_hbm, row_v)
        pltpu.sync_copy(row_v, acc.at[pl.ds(r0, rpt)])
        plsc.subcore_barrier()

        pltpu.async_copy(t_hbm.at[src_v.at[0]], buf_a, sem_a)

        def body(i, carry):
            j = 2 * i
            pltpu.async_copy(t_hbm.at[src_v.at[j + 1]], buf_b, sem_b)
            pltpu.make_async_copy(t_hbm.at[src_v.at[j]], buf_a, sem_a).wait()
            pltpu.sync_copy(buf_a, acc.at[dst_v.at[j]], add=True)
            pltpu.async_copy(t_hbm.at[src_v.at[j + 2]], buf_a, sem_a)
            pltpu.make_async_copy(t_hbm.at[src_v.at[j + 1]], buf_b, sem_b).wait()
            pltpu.sync_copy(buf_b, acc.at[dst_v.at[j + 1]], add=True)
            return carry

        lax.fori_loop(0, KL // 2, body, 0)
        pltpu.make_async_copy(t_hbm.at[src_v.at[KL]], buf_a, sem_a).wait()
        plsc.subcore_barrier()
        pltpu.sync_copy(acc.at[pl.ds(r0, rpt)], row_v)
        pltpu.sync_copy(row_v, out_hbm.at[c, pl.ds(r0, rpt)])

    return prop


def _make_count(n_pad, KL):
    C = 16
    rpt = n_pad // _NS
    kb = KL + 2
    mesh = plsc.VectorSubcoreMesh(
        core_axis_name="c", subcore_axis_name="s",
        num_cores=_NC, num_subcores=_NS)

    @functools.partial(
        pl.kernel,
        out_type=jax.ShapeDtypeStruct((_NC, n_pad, C), jnp.float32),
        mesh=mesh,
        compiler_params=pltpu.CompilerParams(use_tc_tiling_on_sc=False),
        scratch_types=[
            pltpu.VMEM((kb, _EB), jnp.int32),
            pltpu.VMEM((_EB, C), jnp.float32),
            pltpu.VMEM((rpt, C), jnp.float32),
            pltpu.VMEM_SHARED((n_pad, C), jnp.float32),
        ],
    )
    def cnt(ones_hbm, dst_hbm, zeros_hbm, out_hbm, dst_v, buf, row_v, acc):
        c = lax.axis_index("c")
        s = lax.axis_index("s")
        w = s * _NC + c
        pltpu.sync_copy(dst_hbm.at[w], dst_v)
        pltpu.sync_copy(ones_hbm, buf)
        r0 = s * rpt
        pltpu.sync_copy(zeros_hbm, row_v)
        pltpu.sync_copy(row_v, acc.at[pl.ds(r0, rpt)])
        plsc.subcore_barrier()

        def body(j, carry):
            pltpu.sync_copy(buf, acc.at[dst_v.at[j]], add=True)
            return carry

        lax.fori_loop(0, KL, body, 0)
        plsc.subcore_barrier()
        pltpu.sync_copy(acc.at[pl.ds(r0, rpt)], row_v)
        pltpu.sync_copy(row_v, out_hbm.at[c, pl.ds(r0, rpt)])

    return cnt


def _dinv_scale(c0, c1, p2, n_pad):
    rows = n_pad // 8

    def kern(a_ref, b_ref, p2_ref, d_ref, t_ref):
        d = lax.rsqrt(a_ref[...][:, :1] + b_ref[...][:, :1] + 1.0)
        d_ref[...] = d
        t_ref[...] = p2_ref[...] * d

    spec = pl.BlockSpec((rows, 16), lambda i: (i, 0))
    return pl.pallas_call(
        kern,
        grid=(8,),
        in_specs=[spec, spec, pl.BlockSpec((rows, 64), lambda i: (i, 0))],
        out_specs=[
            pl.BlockSpec((rows, 1), lambda i: (i, 0)),
            pl.BlockSpec((rows, 64), lambda i: (i, 0)),
        ],
        out_shape=[
            jax.ShapeDtypeStruct((n_pad, 1), jnp.float32),
            jax.ShapeDtypeStruct((n_pad, 64), jnp.float32),
        ],
    )(c0, c1, p2)


def _mm_split(h, wc):
    n_pad, d = h.shape
    rows = n_pad // 8

    def kern(h_ref, w_ref, p1_ref, p2_ref):
        p = jnp.dot(h_ref[...], w_ref[...], preferred_element_type=jnp.float32)
        p1_ref[...] = p[:, :32]
        p2_ref[...] = p[:, 32:]

    return pl.pallas_call(
        kern,
        grid=(8,),
        in_specs=[
            pl.BlockSpec((rows, d), lambda i: (i, 0)),
            pl.BlockSpec((d, 96), lambda i: (0, 0)),
        ],
        out_specs=[
            pl.BlockSpec((rows, 32), lambda i: (i, 0)),
            pl.BlockSpec((rows, 64), lambda i: (i, 0)),
        ],
        out_shape=[
            jax.ShapeDtypeStruct((n_pad, 32), jnp.float32),
            jax.ShapeDtypeStruct((n_pad, 64), jnp.float32),
        ],
    )(h, wc)


def _mid(pp, t, dinv):
    n_pad = t.shape[0]
    rows = n_pad // 8

    def kern(pp_ref, t_ref, d_ref, u1_ref, t2_ref):
        d = d_ref[...]
        u = (pp_ref[0] + pp_ref[1] + t_ref[...]) * d
        u1_ref[...] = u[:, :32]
        t2_ref[...] = u[:, 32:] * d

    return pl.pallas_call(
        kern,
        grid=(8,),
        in_specs=[
            pl.BlockSpec((2, rows, 64), lambda i: (0, i, 0)),
            pl.BlockSpec((rows, 64), lambda i: (i, 0)),
            pl.BlockSpec((rows, 1), lambda i: (i, 0)),
        ],
        out_specs=[
            pl.BlockSpec((rows, 32), lambda i: (i, 0)),
            pl.BlockSpec((rows, 32), lambda i: (i, 0)),
        ],
        out_shape=[
            jax.ShapeDtypeStruct((n_pad, 32), jnp.float32),
            jax.ShapeDtypeStruct((n_pad, 32), jnp.float32),
        ],
    )(pp, t, dinv)


def _fin_mm(qq, t2, p1, u1, bc, dinv, wc_next):
    n_pad = t2.shape[0]
    rows = n_pad // 8

    def kern(qq_ref, t2_ref, p1_ref, u1_ref, b_ref, d_ref, w_ref,
             p1n_ref, tn_ref):
        d = d_ref[...]
        v = (qq_ref[0] + qq_ref[1] + t2_ref[...]) * d
        cat = jnp.concatenate([p1_ref[...], u1_ref[...], v], axis=1)
        h = jnp.tanh(cat + b_ref[...])
        p = jnp.dot(h, w_ref[...], preferred_element_type=jnp.float32)
        p1n_ref[...] = p[:, :32]
        tn_ref[...] = p[:, 32:] * d

    spec32 = pl.BlockSpec((rows, 32), lambda i: (i, 0))
    return pl.pallas_call(
        kern,
        grid=(8,),
        in_specs=[
            pl.BlockSpec((2, rows, 32), lambda i: (0, i, 0)),
            spec32, spec32, spec32,
            pl.BlockSpec((1, 96), lambda i: (0, 0)),
            pl.BlockSpec((rows, 1), lambda i: (i, 0)),
            pl.BlockSpec((96, 96), lambda i: (0, 0)),
        ],
        out_specs=[
            pl.BlockSpec((rows, 32), lambda i: (i, 0)),
            pl.BlockSpec((rows, 64), lambda i: (i, 0)),
        ],
        out_shape=[
            jax.ShapeDtypeStruct((n_pad, 32), jnp.float32),
            jax.ShapeDtypeStruct((n_pad, 64), jnp.float32),
        ],
    )(qq, t2, p1, u1, bc, dinv, wc_next)


def _fin(qq, t2, p1, u1, bc, dinv):
    n_pad = t2.shape[0]
    rows = n_pad // 8

    def kern(qq_ref, t2_ref, p1_ref, u1_ref, b_ref, d_ref, o_ref):
        v = (qq_ref[0] + qq_ref[1] + t2_ref[...]) * d_ref[...]
        cat = jnp.concatenate([p1_ref[...], u1_ref[...], v], axis=1)
        o_ref[...] = jnp.tanh(cat + b_ref[...])

    spec32 = pl.BlockSpec((rows, 32), lambda i: (i, 0))
    return pl.pallas_call(
        kern,
        grid=(8,),
        in_specs=[
            pl.BlockSpec((2, rows, 32), lambda i: (0, i, 0)),
            spec32, spec32, spec32,
            pl.BlockSpec((1, 96), lambda i: (0, 0)),
            pl.BlockSpec((rows, 1), lambda i: (i, 0)),
        ],
        out_specs=pl.BlockSpec((rows, 96), lambda i: (i, 0)),
        out_shape=jax.ShapeDtypeStruct((n_pad, 96), jnp.float32),
    )(qq, t2, p1, u1, bc, dinv)


def kernel(x, params, edge_index):
    n, d_in = x.shape
    e = edge_index.shape[1]
    n_pad = -(-(n + 1) // (_NS * 8)) * (_NS * 8)
    k = -(-e // (_NW * _EB))
    kl = k + (k % 2)
    e_pad = kl * _NW * _EB
    pad_rows = n_pad - n

    pad_src = jnp.arange(e_pad - e, dtype=jnp.int32) % n
    pad_dst = n + jnp.arange(e_pad - e, dtype=jnp.int32) % pad_rows
    src = jnp.concatenate(
        [edge_index[0], pad_src]).reshape(_NW, kl, _EB)
    dst = jnp.concatenate(
        [edge_index[1], pad_dst]).reshape(_NW, kl, _EB)
    la_src = (jnp.arange(2 * _EB, dtype=jnp.int32) % n).reshape(1, 2, _EB)
    la_dst = (n + jnp.arange(2 * _EB, dtype=jnp.int32) % pad_rows).reshape(1, 2, _EB)
    src = jnp.concatenate(
        [src, jnp.broadcast_to(la_src, (_NW, 2, _EB))], axis=1)
    dst = jnp.concatenate(
        [dst, jnp.broadcast_to(la_dst, (_NW, 2, _EB))], axis=1)

    rpt = n_pad // _NS
    z16 = jnp.zeros((rpt, 16), jnp.float32)
    z32 = jnp.zeros((rpt, 32), jnp.float32)
    z64 = jnp.zeros((rpt, 64), jnp.float32)
    prop32 = _make_propagate(n_pad, 32, kl)
    prop64 = _make_propagate(n_pad, 64, kl)

    cnt = _make_count(n_pad, kl)(jnp.ones((_EB, 16), jnp.float32), dst, z16)

    x_pad = jnp.concatenate(
        [x, jnp.zeros((pad_rows, d_in), jnp.float32)], axis=0)
    wcs = [jnp.concatenate(params["W" + l], axis=1) for l in ("1", "2", "3")]
    bcs = [jnp.concatenate(params["b" + l]).reshape(1, 96) for l in ("1", "2", "3")]

    p1, p2 = _mm_split(x_pad, wcs[0])
    dinv, t = _dinv_scale(cnt[0], cnt[1], p2, n_pad)
    for li in range(3):
        pp = prop64(t, src, dst, z64)
        u1, t2 = _mid(pp, t, dinv)
        qq = prop32(t2, src, dst, z32)
        if li < 2:
            p1, t = _fin_mm(qq, t2, p1, u1, bcs[li], dinv, wcs[li + 1])
        else:
            h = _fin(qq, t2, p1, u1, bcs[li], dinv)
    return h[:n]
